# baseline (speedup 1.0000x reference)
import jax
import jax.numpy as jnp
from jax.experimental import pallas as pl
from jax.experimental.pallas import tpu as pltpu

VOCAB = 10
DIM = 10
TOK = 200
ROWS = 16384
BT = 40


CH = 512


def _body(x_ref, w1_ref, w2_ref, out_ref):
    ws = [[w1_ref[v, d] + w2_ref[v, d] for d in range(DIM)]
          for v in range(VOCAB)]
    for c in range(ROWS // CH):
        sl = slice(c * CH, (c + 1) * CH)
        xc = x_ref[:, sl]
        b0 = (xc & 1) != 0
        b1 = (xc & 2) != 0
        b2 = (xc & 4) != 0
        b3 = xc >= 8
        for d in range(DIM):
            s01 = jnp.where(b0, ws[1][d], ws[0][d])
            s23 = jnp.where(b0, ws[3][d], ws[2][d])
            s45 = jnp.where(b0, ws[5][d], ws[4][d])
            s67 = jnp.where(b0, ws[7][d], ws[6][d])
            s89 = jnp.where(b0, ws[9][d], ws[8][d])
            t03 = jnp.where(b1, s23, s01)
            t47 = jnp.where(b1, s67, s45)
            u07 = jnp.where(b2, t47, t03)
            out_ref[d, :, sl] = jnp.where(b3, s89, u07)


@jax.jit
def kernel(x, W1, W2):
    xt = x.T
    outt = pl.pallas_call(
        _body,
        grid=(TOK // BT,),
        in_specs=[
            pl.BlockSpec((BT, ROWS), lambda i: (i, 0)),
            pl.BlockSpec(memory_space=pltpu.SMEM),
            pl.BlockSpec(memory_space=pltpu.SMEM),
        ],
        out_specs=pl.BlockSpec((DIM, BT, ROWS), lambda i: (0, i, 0)),
        out_shape=jax.ShapeDtypeStruct((DIM, TOK, ROWS), jnp.float32),
    )(xt, W1, W2)
    return outt.transpose(2, 1, 0)
